# baseline (speedup 1.0000x reference)
import functools
import math

import jax
import jax.numpy as jnp
from jax import lax
from jax.experimental import pallas as pl
from jax.experimental.pallas import tpu as pltpu
from jax.experimental.pallas import tpu_sc as plsc

_DT_SCALE = math.log1p(168.0)

_B, _L = 16384, 200
_N = _B * _L
_NC, _NS, _LANES = 2, 16, 16
_NW = _NC * _NS
_PER_W = _N // _NW
_CH = 12800
_NCHUNKS = _PER_W // _CH

_mesh = plsc.VectorSubcoreMesh(core_axis_name="c", subcore_axis_name="s")

_GATHER_DNUMS = lax.GatherDimensionNumbers(
    offset_dims=(), collapsed_slice_dims=(0,), start_index_map=(0,))


@functools.partial(
    pl.kernel,
    mesh=_mesh,
    out_type=jax.ShapeDtypeStruct((_N,), jnp.float32),
    scratch_types=[
        pltpu.VMEM((_LANES,), jnp.float32),
        pltpu.VMEM((_CH,), jnp.float32),
        pltpu.VMEM((_CH,), jnp.float32),
        pltpu.VMEM((_CH,), jnp.int32),
        pltpu.VMEM((_CH,), jnp.float32),
    ],
)
def _recovery(mpc_hbm, dt_hbm, idx_hbm, tab_hbm, out_hbm,
              tab_v, mpc_v, dt_v, idx_v, out_v):
    wid = lax.axis_index("s") * _NC + lax.axis_index("c")
    base = wid * _PER_W

    pltpu.sync_copy(tab_hbm, tab_v)
    tab_vec = -jnp.exp(-tab_v[...])

    for ci in range(_NCHUNKS):
        off = base + ci * _CH
        pltpu.sync_copy(mpc_hbm.at[pl.ds(off, _CH)], mpc_v)
        pltpu.sync_copy(dt_hbm.at[pl.ds(off, _CH)], dt_v)
        pltpu.sync_copy(idx_hbm.at[pl.ds(off, _CH)], idx_v)

        def body(i, _):
            sl = pl.ds(i * _LANES, _LANES)
            neg_inv_tau = lax.gather(
                tab_vec, idx_v[sl][:, None], _GATHER_DNUMS, (1,),
                mode=lax.GatherScatterMode.PROMISE_IN_BOUNDS)
            dt_hours = jnp.exp(dt_v[sl] * _DT_SCALE) - 1.0
            decay = jnp.exp(dt_hours * neg_inv_tau)
            out_v[sl] = 1.0 - (1.0 - mpc_v[sl]) * decay
            return 0

        lax.fori_loop(0, _CH // _LANES, body, 0)

        pltpu.sync_copy(out_v, out_hbm.at[pl.ds(off, _CH)])


def kernel(mpc, delta_t, muscle_idx, log_tau):
    mpc_f = mpc.reshape(-1)
    dt_f = delta_t.reshape(-1)
    idx_f = muscle_idx.reshape(-1).astype(jnp.int32)
    tab = jnp.pad(log_tau.astype(jnp.float32), (0, _LANES - log_tau.shape[0]))
    out = _recovery(mpc_f, dt_f, idx_f, tab)
    return out.reshape(mpc.shape)


# speedup vs baseline: 1.0690x; 1.0690x over previous
import functools
import math

import jax
import jax.numpy as jnp
from jax import lax
from jax.experimental import pallas as pl
from jax.experimental.pallas import tpu as pltpu
from jax.experimental.pallas import tpu_sc as plsc

_DT_SCALE = math.log1p(168.0)

_B, _L = 16384, 200
_N = _B * _L
_NC, _NS, _LANES = 2, 16, 16
_NW = _NC * _NS
_PER_W = _N // _NW
_CH = 12800
_NCHUNKS = _PER_W // _CH

_mesh = plsc.VectorSubcoreMesh(core_axis_name="c", subcore_axis_name="s")

_GATHER_DNUMS = lax.GatherDimensionNumbers(
    offset_dims=(), collapsed_slice_dims=(0,), start_index_map=(0,))


@functools.partial(
    pl.kernel,
    mesh=_mesh,
    out_type=jax.ShapeDtypeStruct((_N,), jnp.float32),
    scratch_types=[
        pltpu.VMEM((_LANES,), jnp.float32),
        pltpu.VMEM((_CH,), jnp.float32),
        pltpu.VMEM((_CH,), jnp.float32),
        pltpu.VMEM((_CH,), jnp.int32),
        pltpu.VMEM((_CH,), jnp.float32),
    ],
)
def _recovery(mpc_hbm, dt_hbm, idx_hbm, tab_hbm, out_hbm,
              tab_v, mpc_v, dt_v, idx_v, out_v):
    wid = lax.axis_index("s") * _NC + lax.axis_index("c")
    base = wid * _PER_W

    pltpu.sync_copy(tab_hbm, tab_v)
    tab_vec = -jnp.exp(-tab_v[...])

    for ci in range(_NCHUNKS):
        off = base + ci * _CH
        pltpu.sync_copy(mpc_hbm.at[pl.ds(off, _CH)], mpc_v)
        pltpu.sync_copy(dt_hbm.at[pl.ds(off, _CH)], dt_v)
        pltpu.sync_copy(idx_hbm.at[pl.ds(off, _CH)], idx_v)

        @plsc.parallel_loop(0, _CH, _LANES, unroll=8)
        def body(i):
            sl = pl.ds(i, _LANES)
            neg_inv_tau = lax.gather(
                tab_vec, idx_v[sl][:, None], _GATHER_DNUMS, (1,),
                mode=lax.GatherScatterMode.PROMISE_IN_BOUNDS)
            dt_hours = jnp.exp(dt_v[sl] * _DT_SCALE) - 1.0
            decay = jnp.exp(dt_hours * neg_inv_tau)
            out_v[sl] = 1.0 - (1.0 - mpc_v[sl]) * decay

        pltpu.sync_copy(out_v, out_hbm.at[pl.ds(off, _CH)])


def kernel(mpc, delta_t, muscle_idx, log_tau):
    mpc_f = mpc.reshape(-1)
    dt_f = delta_t.reshape(-1)
    idx_f = muscle_idx.reshape(-1).astype(jnp.int32)
    tab = jnp.pad(log_tau.astype(jnp.float32), (0, _LANES - log_tau.shape[0]))
    out = _recovery(mpc_f, dt_f, idx_f, tab)
    return out.reshape(mpc.shape)


# speedup vs baseline: 1.5981x; 1.4950x over previous
import functools
import math

import jax
import jax.numpy as jnp
from jax import lax
from jax.experimental import pallas as pl
from jax.experimental.pallas import tpu as pltpu
from jax.experimental.pallas import tpu_sc as plsc

_DT_SCALE = math.log1p(168.0)

_B, _L = 16384, 200
_NC, _NS, _LANES = 2, 16, 16
_NW = _NC * _NS
_ROWS_W = _B // _NW
_CR = 64
_NCHUNKS = _ROWS_W // _CR
_COFFS = tuple(range(0, 192, 16)) + (184,)

_mesh = plsc.VectorSubcoreMesh(core_axis_name="c", subcore_axis_name="s")

_GATHER_DNUMS = lax.GatherDimensionNumbers(
    offset_dims=(), collapsed_slice_dims=(0,), start_index_map=(0,))


@functools.partial(
    pl.kernel,
    mesh=_mesh,
    out_type=jax.ShapeDtypeStruct((_B, _L), jnp.float32),
    scratch_types=[
        pltpu.VMEM((_LANES,), jnp.float32),
        pltpu.VMEM((_CR, _L), jnp.float32),
        pltpu.VMEM((_CR, _L), jnp.float32),
        pltpu.VMEM((_CR, _L), jnp.int32),
        pltpu.VMEM((_CR, _L), jnp.float32),
    ],
)
def _recovery(mpc_hbm, dt_hbm, idx_hbm, tab_hbm, out_hbm,
              tab_v, mpc_v, dt_v, idx_v, out_v):
    wid = lax.axis_index("s") * _NC + lax.axis_index("c")
    row0 = wid * _ROWS_W

    pltpu.sync_copy(tab_hbm, tab_v)
    tab_vec = -jnp.exp(-tab_v[...])

    for ci in range(_NCHUNKS):
        r0 = row0 + ci * _CR
        pltpu.sync_copy(mpc_hbm.at[pl.ds(r0, _CR)], mpc_v)
        pltpu.sync_copy(dt_hbm.at[pl.ds(r0, _CR)], dt_v)
        pltpu.sync_copy(idx_hbm.at[pl.ds(r0, _CR)], idx_v)

        @plsc.parallel_loop(0, _CR, 1, unroll=2)
        def body(r):
            for c in _COFFS:
                sl = pl.ds(c, _LANES)
                neg_inv_tau = lax.gather(
                    tab_vec, idx_v[r, sl][:, None], _GATHER_DNUMS, (1,),
                    mode=lax.GatherScatterMode.PROMISE_IN_BOUNDS)
                dt_hours = jnp.exp(dt_v[r, sl] * _DT_SCALE) - 1.0
                decay = jnp.exp(dt_hours * neg_inv_tau)
                out_v[r, sl] = 1.0 - (1.0 - mpc_v[r, sl]) * decay

        pltpu.sync_copy(out_v, out_hbm.at[pl.ds(r0, _CR)])


def kernel(mpc, delta_t, muscle_idx, log_tau):
    idx = muscle_idx.astype(jnp.int32)
    tab = jnp.pad(log_tau.astype(jnp.float32), (0, _LANES - log_tau.shape[0]))
    return _recovery(mpc, delta_t, idx, tab)
